# baseline (speedup 1.0000x reference)
import functools

import jax
import jax.numpy as jnp
from jax import lax
from jax.experimental import pallas as pl
from jax.experimental.pallas import tpu as pltpu

NUM_EXPERTS = 8
HIDDEN = 1024
INTER = 1408
GROUP = 128
TOKENS = 2048
TOPK = 2

HID_W = HIDDEN // 8
INT_W = INTER // 8

NPAIRS = TOKENS * TOPK
TILE = 128
NT = NPAIRS // TILE + NUM_EXPERTS
ROWS = NT * TILE


def _decode_nibbles(nib):
    m = nib & 7
    e = m >> 1
    f = (m & 1).astype(jnp.float32)
    pow2 = (jnp.int32(1) << e).astype(jnp.float32) * 0.5
    mag = jnp.where(e == 0, 0.5 * f, pow2 * (1.0 + 0.5 * f))
    sign = 1.0 - 2.0 * (nib >> 3).astype(jnp.float32)
    return sign * mag


def _unpack(packed_u32, scale_rep, w_scratch, n_words):
    srep = scale_rep.astype(jnp.float32)
    for k in range(8):
        nib = lax.shift_right_logical(
            packed_u32, jnp.uint32(4 * k)).astype(jnp.int32) & 15
        val = _decode_nibbles(nib) * srep
        w_scratch[:, k * n_words:(k + 1) * n_words] = val.astype(jnp.bfloat16)


def _gmm_kernel(eid_ref, xs_ref, wrow_ref,
                gp_ref, gs_ref, up_ref, us_ref, dp_ref, ds_ref,
                y_ref, wg_ref, wu_ref, wd_ref, prev_ref):
    i = pl.program_id(0)
    eid = eid_ref[i]

    @pl.when((i == 0) | (eid != prev_ref[0]))
    def _():
        _unpack(gp_ref[eid], gs_ref[eid], wg_ref, HID_W)
        _unpack(up_ref[eid], us_ref[eid], wu_ref, HID_W)
        _unpack(dp_ref[eid], ds_ref[eid], wd_ref, INT_W)
        prev_ref[0] = eid

    x = xs_ref[...].astype(jnp.bfloat16)
    dn = (((1,), (1,)), ((), ()))
    g = lax.dot_general(x, wg_ref[...], dn, preferred_element_type=jnp.float32)
    u = lax.dot_general(x, wu_ref[...], dn, preferred_element_type=jnp.float32)
    h = (g * jax.nn.sigmoid(g) * u * wrow_ref[...]).astype(jnp.bfloat16)
    y_ref[...] = lax.dot_general(h, wd_ref[...], dn,
                                 preferred_element_type=jnp.float32)


def _perm_inter(a):
    E = a.shape[0]
    return a.reshape(E, INT_W, 8, *a.shape[2:]).swapaxes(1, 2).reshape(
        E, INTER, *a.shape[2:])


def _routing_setup(top_k_index, top_k_weights):
    i32 = jnp.int32
    tki_f = top_k_index.reshape(-1)
    tkw_f = top_k_weights.reshape(-1)
    order = jnp.argsort(tki_f, stable=True)
    sorted_e = tki_f[order]
    counts = jnp.bincount(tki_f, length=NUM_EXPERTS)
    padded = ((counts + TILE - 1) // TILE) * TILE
    cum_pad = jnp.cumsum(padded)
    pad_off = cum_pad - padded
    raw_off = jnp.cumsum(counts) - counts
    rank = jnp.arange(NPAIRS, dtype=i32) - raw_off[sorted_e].astype(i32)
    pos = pad_off[sorted_e].astype(i32) + rank
    row_ids = jnp.zeros(ROWS, i32).at[pos].set(
        (order // TOPK).astype(i32))
    wrow = jnp.zeros((ROWS, 1), jnp.float32).at[pos, 0].set(tkw_f[order])
    pos_nat = jnp.zeros(NPAIRS, i32).at[order].set(pos)
    p0 = pos_nat[0::TOPK]
    p1 = pos_nat[1::TOPK]
    tile_eid = jnp.clip(
        jnp.searchsorted(cum_pad, jnp.arange(NT, dtype=i32) * TILE,
                         side='right'),
        0, NUM_EXPERTS - 1).astype(i32)
    return row_ids, wrow, p0, p1, tile_eid


@jax.jit
def kernel(hidden_states, top_k_index, top_k_weights,
           gate_proj_packed, gate_proj_scales,
           up_proj_packed, up_proj_scales,
           down_proj_packed, down_proj_scales):
    xr = hidden_states.reshape(TOKENS, HID_W, 8).swapaxes(1, 2).reshape(
        TOKENS, HIDDEN)

    gp = _perm_inter(gate_proj_packed)
    up = _perm_inter(up_proj_packed)
    gs = jnp.repeat(_perm_inter(gate_proj_scales.transpose(0, 2, 1)), 16,
                    axis=2).astype(jnp.bfloat16)
    us = jnp.repeat(_perm_inter(up_proj_scales.transpose(0, 2, 1)), 16,
                    axis=2).astype(jnp.bfloat16)
    ds = jnp.repeat(down_proj_scales.transpose(0, 2, 1), 16,
                    axis=2).astype(jnp.bfloat16)

    row_ids, wrow, p0, p1, tile_eid = _routing_setup(top_k_index,
                                                     top_k_weights)

    xs = jnp.take(xr, row_ids, axis=0)

    resident = lambda s: pl.BlockSpec(s, lambda i: (0, 0, 0))
    y = pl.pallas_call(
        _gmm_kernel,
        grid=(NT,),
        in_specs=[
            pl.BlockSpec(memory_space=pltpu.SMEM),
            pl.BlockSpec((TILE, HIDDEN), lambda i: (i, 0)),
            pl.BlockSpec((TILE, 1), lambda i: (i, 0)),
            resident((NUM_EXPERTS, INTER, HID_W)),
            resident((NUM_EXPERTS, INTER, HID_W)),
            resident((NUM_EXPERTS, INTER, HID_W)),
            resident((NUM_EXPERTS, INTER, HID_W)),
            resident((NUM_EXPERTS, HIDDEN, INT_W)),
            resident((NUM_EXPERTS, HIDDEN, INT_W)),
        ],
        out_specs=pl.BlockSpec((TILE, HIDDEN), lambda i: (i, 0)),
        out_shape=jax.ShapeDtypeStruct((ROWS, HIDDEN), jnp.float32),
        scratch_shapes=[
            pltpu.VMEM((INTER, HIDDEN), jnp.bfloat16),
            pltpu.VMEM((INTER, HIDDEN), jnp.bfloat16),
            pltpu.VMEM((HIDDEN, INTER), jnp.bfloat16),
            pltpu.SMEM((1,), jnp.int32),
        ],
    )(tile_eid, xs, wrow, gp, gs, up, us, down_proj_packed, ds)

    out = jnp.take(y, p0, axis=0) + jnp.take(y, p1, axis=0)
    return out


# speedup vs baseline: 1.0493x; 1.0493x over previous
import functools

import jax
import jax.numpy as jnp
from jax import lax
from jax.experimental import pallas as pl
from jax.experimental.pallas import tpu as pltpu

NUM_EXPERTS = 8
HIDDEN = 1024
INTER = 1408
GROUP = 128
TOKENS = 2048
TOPK = 2

HID_W = HIDDEN // 8
INT_W = INTER // 8

NPAIRS = TOKENS * TOPK
TILE = 128
NT = NPAIRS // TILE + NUM_EXPERTS
ROWS = NT * TILE


def _decode_nibbles(nib):
    m = nib & 7
    e = m >> 1
    f = (m & 1).astype(jnp.float32)
    pow2 = (jnp.int32(1) << e).astype(jnp.float32) * 0.5
    mag = jnp.where(e == 0, 0.5 * f, pow2 * (1.0 + 0.5 * f))
    sign = 1.0 - 2.0 * (nib >> 3).astype(jnp.float32)
    return sign * mag


def _unpack(packed_u32, scale_rep, w_scratch, n_words):
    srep = scale_rep.astype(jnp.float32)
    for k in range(8):
        nib = lax.shift_right_logical(
            packed_u32, jnp.uint32(4 * k)).astype(jnp.int32) & 15
        val = _decode_nibbles(nib) * srep
        w_scratch[:, k * n_words:(k + 1) * n_words] = val.astype(jnp.bfloat16)


def _gmm_kernel(eid_ref, xs_ref, wrow_ref,
                gp_ref, gs_ref, up_ref, us_ref, dp_ref, ds_ref,
                y_ref, wg_ref, wu_ref, wd_ref, prev_ref):
    i = pl.program_id(0)
    eid = eid_ref[i]

    @pl.when((i == 0) | (eid != prev_ref[0]))
    def _():
        _unpack(gp_ref[eid], gs_ref[eid], wg_ref, HID_W)
        _unpack(up_ref[eid], us_ref[eid], wu_ref, HID_W)
        _unpack(dp_ref[eid], ds_ref[eid], wd_ref, INT_W)
        prev_ref[0] = eid

    x = xs_ref[...].astype(jnp.bfloat16)
    dn = (((1,), (1,)), ((), ()))
    g = lax.dot_general(x, wg_ref[...], dn, preferred_element_type=jnp.float32)
    u = lax.dot_general(x, wu_ref[...], dn, preferred_element_type=jnp.float32)
    h = (g * jax.nn.sigmoid(g) * u * wrow_ref[...]).astype(jnp.bfloat16)
    y_ref[...] = lax.dot_general(h, wd_ref[...], dn,
                                 preferred_element_type=jnp.float32)


def _perm_inter(a):
    E = a.shape[0]
    return a.reshape(E, INT_W, 8, *a.shape[2:]).swapaxes(1, 2).reshape(
        E, INTER, *a.shape[2:])


def _routing_setup(top_k_index, top_k_weights):
    i32 = jnp.int32
    tki_f = top_k_index.reshape(-1)
    tkw_f = top_k_weights.reshape(-1)
    onehot = (tki_f[:, None] == jnp.arange(NUM_EXPERTS, dtype=i32)[None, :]
              ).astype(i32)
    rank_all = jnp.cumsum(onehot, axis=0) - onehot
    counts = jnp.sum(onehot, axis=0)
    padded = ((counts + TILE - 1) // TILE) * TILE
    cum_pad = jnp.cumsum(padded)
    pad_off = cum_pad - padded
    rank = jnp.take_along_axis(rank_all, tki_f[:, None], axis=1)[:, 0]
    pos = pad_off[tki_f].astype(i32) + rank.astype(i32)
    pair_rows = (jnp.arange(NPAIRS, dtype=i32) // TOPK)
    row_ids = jnp.zeros(ROWS, i32).at[pos].set(pair_rows)
    wrow = jnp.zeros((ROWS, 1), jnp.float32).at[pos, 0].set(tkw_f)
    p0 = pos[0::TOPK]
    p1 = pos[1::TOPK]
    tile_eid = jnp.clip(
        jnp.searchsorted(cum_pad, jnp.arange(NT, dtype=i32) * TILE,
                         side='right'),
        0, NUM_EXPERTS - 1).astype(i32)
    return row_ids, wrow, p0, p1, tile_eid


@jax.jit
def kernel(hidden_states, top_k_index, top_k_weights,
           gate_proj_packed, gate_proj_scales,
           up_proj_packed, up_proj_scales,
           down_proj_packed, down_proj_scales):
    xr = hidden_states.reshape(TOKENS, HID_W, 8).swapaxes(1, 2).reshape(
        TOKENS, HIDDEN)

    gp = _perm_inter(gate_proj_packed)
    up = _perm_inter(up_proj_packed)
    gs = jnp.repeat(_perm_inter(gate_proj_scales.transpose(0, 2, 1)), 16,
                    axis=2).astype(jnp.bfloat16)
    us = jnp.repeat(_perm_inter(up_proj_scales.transpose(0, 2, 1)), 16,
                    axis=2).astype(jnp.bfloat16)
    ds = jnp.repeat(down_proj_scales.transpose(0, 2, 1), 16,
                    axis=2).astype(jnp.bfloat16)

    row_ids, wrow, p0, p1, tile_eid = _routing_setup(top_k_index,
                                                     top_k_weights)

    xs = jnp.take(xr, row_ids, axis=0)

    resident = lambda s: pl.BlockSpec(s, lambda i: (0, 0, 0))
    y = pl.pallas_call(
        _gmm_kernel,
        grid=(NT,),
        in_specs=[
            pl.BlockSpec(memory_space=pltpu.SMEM),
            pl.BlockSpec((TILE, HIDDEN), lambda i: (i, 0)),
            pl.BlockSpec((TILE, 1), lambda i: (i, 0)),
            resident((NUM_EXPERTS, INTER, HID_W)),
            resident((NUM_EXPERTS, INTER, HID_W)),
            resident((NUM_EXPERTS, INTER, HID_W)),
            resident((NUM_EXPERTS, INTER, HID_W)),
            resident((NUM_EXPERTS, HIDDEN, INT_W)),
            resident((NUM_EXPERTS, HIDDEN, INT_W)),
        ],
        out_specs=pl.BlockSpec((TILE, HIDDEN), lambda i: (i, 0)),
        out_shape=jax.ShapeDtypeStruct((ROWS, HIDDEN), jnp.float32),
        scratch_shapes=[
            pltpu.VMEM((INTER, HIDDEN), jnp.bfloat16),
            pltpu.VMEM((INTER, HIDDEN), jnp.bfloat16),
            pltpu.VMEM((HIDDEN, INTER), jnp.bfloat16),
            pltpu.SMEM((1,), jnp.int32),
        ],
    )(tile_eid, xs, wrow, gp, gs, up, us, down_proj_packed, ds)

    out = jnp.take(y, p0, axis=0) + jnp.take(y, p1, axis=0)
    return out


# speedup vs baseline: 1.2189x; 1.1617x over previous
import functools

import jax
import jax.numpy as jnp
from jax import lax
from jax.experimental import pallas as pl
from jax.experimental.pallas import tpu as pltpu

NUM_EXPERTS = 8
HIDDEN = 1024
INTER = 1408
GROUP = 128
TOKENS = 2048
TOPK = 2

HID_W = HIDDEN // 8
INT_W = INTER // 8

NPAIRS = TOKENS * TOPK
TILE = 256
NT = NPAIRS // TILE + NUM_EXPERTS
ROWS = NT * TILE


def _decode_nibbles(nib):
    m = nib & 7
    e = m >> 1
    f = (m & 1).astype(jnp.float32)
    pow2 = (jnp.int32(1) << e).astype(jnp.float32) * 0.5
    mag = jnp.where(e == 0, 0.5 * f, pow2 * (1.0 + 0.5 * f))
    sign = 1.0 - 2.0 * (nib >> 3).astype(jnp.float32)
    return sign * mag


def _unpack(packed_u32, scale_rep, w_scratch, n_words):
    srep = scale_rep.astype(jnp.float32)
    for k in range(8):
        nib = lax.shift_right_logical(
            packed_u32, jnp.uint32(4 * k)).astype(jnp.int32) & 15
        val = _decode_nibbles(nib) * srep
        w_scratch[:, k * n_words:(k + 1) * n_words] = val.astype(jnp.bfloat16)


def _gmm_kernel(eid_ref, act_ref, xs_ref, wrow_ref,
                gp_ref, gs_ref, up_ref, us_ref, dp_ref, ds_ref,
                y_ref, wg_ref, wu_ref, wd_ref, prev_ref):
    i = pl.program_id(0)
    eid = eid_ref[i]

    @pl.when(act_ref[i] == 1)
    def _():
        @pl.when((i == 0) | (eid != prev_ref[0]))
        def _():
            _unpack(gp_ref[eid], gs_ref[eid], wg_ref, HID_W)
            _unpack(up_ref[eid], us_ref[eid], wu_ref, HID_W)
            _unpack(dp_ref[eid], ds_ref[eid], wd_ref, INT_W)
            prev_ref[0] = eid

        x = xs_ref[...].astype(jnp.bfloat16)
        dn = (((1,), (1,)), ((), ()))
        g = lax.dot_general(x, wg_ref[...], dn,
                            preferred_element_type=jnp.float32)
        u = lax.dot_general(x, wu_ref[...], dn,
                            preferred_element_type=jnp.float32)
        h = (g * jax.nn.sigmoid(g) * u * wrow_ref[...]).astype(jnp.bfloat16)
        y_ref[...] = lax.dot_general(h, wd_ref[...], dn,
                                     preferred_element_type=jnp.float32)


def _perm_inter(a):
    E = a.shape[0]
    return a.reshape(E, INT_W, 8, *a.shape[2:]).swapaxes(1, 2).reshape(
        E, INTER, *a.shape[2:])


def _routing_setup(top_k_index, top_k_weights):
    i32 = jnp.int32
    tki_f = top_k_index.reshape(-1)
    tkw_f = top_k_weights.reshape(-1)
    onehot = (tki_f[:, None] == jnp.arange(NUM_EXPERTS, dtype=i32)[None, :]
              ).astype(i32)
    rank_all = jnp.cumsum(onehot, axis=0) - onehot
    counts = jnp.sum(onehot, axis=0)
    padded = ((counts + TILE - 1) // TILE) * TILE
    cum_pad = jnp.cumsum(padded)
    pad_off = cum_pad - padded
    rank = jnp.take_along_axis(rank_all, tki_f[:, None], axis=1)[:, 0]
    pos = pad_off[tki_f].astype(i32) + rank.astype(i32)
    pair_rows = (jnp.arange(NPAIRS, dtype=i32) // TOPK)
    row_ids = jnp.zeros(ROWS, i32).at[pos].set(pair_rows)
    wrow = jnp.zeros((ROWS, 1), jnp.float32).at[pos, 0].set(tkw_f)
    p0 = pos[0::TOPK]
    p1 = pos[1::TOPK]
    tile_starts = jnp.arange(NT, dtype=i32) * TILE
    tile_eid = jnp.clip(
        jnp.searchsorted(cum_pad, tile_starts, side='right'),
        0, NUM_EXPERTS - 1).astype(i32)
    tile_act = (tile_starts <
                (pad_off + counts)[tile_eid].astype(i32)).astype(i32)
    return row_ids, wrow, p0, p1, tile_eid, tile_act


@jax.jit
def kernel(hidden_states, top_k_index, top_k_weights,
           gate_proj_packed, gate_proj_scales,
           up_proj_packed, up_proj_scales,
           down_proj_packed, down_proj_scales):
    xr = hidden_states.reshape(TOKENS, HID_W, 8).swapaxes(1, 2).reshape(
        TOKENS, HIDDEN)

    gp = _perm_inter(gate_proj_packed)
    up = _perm_inter(up_proj_packed)
    gs = jnp.repeat(_perm_inter(gate_proj_scales.transpose(0, 2, 1)), 16,
                    axis=2).astype(jnp.bfloat16)
    us = jnp.repeat(_perm_inter(up_proj_scales.transpose(0, 2, 1)), 16,
                    axis=2).astype(jnp.bfloat16)
    ds = jnp.repeat(down_proj_scales.transpose(0, 2, 1), 16,
                    axis=2).astype(jnp.bfloat16)

    row_ids, wrow, p0, p1, tile_eid, tile_act = _routing_setup(
        top_k_index, top_k_weights)

    xs = jnp.take(xr, row_ids, axis=0)

    resident = lambda s: pl.BlockSpec(s, lambda i: (0, 0, 0))
    y = pl.pallas_call(
        _gmm_kernel,
        grid=(NT,),
        in_specs=[
            pl.BlockSpec(memory_space=pltpu.SMEM),
            pl.BlockSpec(memory_space=pltpu.SMEM),
            pl.BlockSpec((TILE, HIDDEN), lambda i: (i, 0)),
            pl.BlockSpec((TILE, 1), lambda i: (i, 0)),
            resident((NUM_EXPERTS, INTER, HID_W)),
            resident((NUM_EXPERTS, INTER, HID_W)),
            resident((NUM_EXPERTS, INTER, HID_W)),
            resident((NUM_EXPERTS, INTER, HID_W)),
            resident((NUM_EXPERTS, HIDDEN, INT_W)),
            resident((NUM_EXPERTS, HIDDEN, INT_W)),
        ],
        out_specs=pl.BlockSpec((TILE, HIDDEN), lambda i: (i, 0)),
        out_shape=jax.ShapeDtypeStruct((ROWS, HIDDEN), jnp.float32),
        scratch_shapes=[
            pltpu.VMEM((INTER, HIDDEN), jnp.bfloat16),
            pltpu.VMEM((INTER, HIDDEN), jnp.bfloat16),
            pltpu.VMEM((HIDDEN, INTER), jnp.bfloat16),
            pltpu.SMEM((1,), jnp.int32),
        ],
    )(tile_eid, tile_act, xs, wrow, gp, gs, up, us, down_proj_packed, ds)

    out = jnp.take(y, p0, axis=0) + jnp.take(y, p1, axis=0)
    return out


# speedup vs baseline: 1.6444x; 1.3491x over previous
import functools

import jax
import jax.numpy as jnp
from jax import lax
from jax.experimental import pallas as pl
from jax.experimental.pallas import tpu as pltpu
from jax.experimental.pallas import tpu_sc as plsc

NUM_EXPERTS = 8
HIDDEN = 1024
INTER = 1408
GROUP = 128
TOKENS = 2048
TOPK = 2

HID_W = HIDDEN // 8
INT_W = INTER // 8


def _decode_nibbles(nib):
    m = nib & 7
    e = m >> 1
    f = (m & 1).astype(jnp.float32)
    pow2 = (jnp.int32(1) << e).astype(jnp.float32) * 0.5
    mag = jnp.where(e == 0, 0.5 * f, pow2 * (1.0 + 0.5 * f))
    sign = 1.0 - 2.0 * (nib >> 3).astype(jnp.float32)
    return sign * mag


def _moe_kernel(tki_ref, tkw_ref, x_ref,
                gp_ref, gs_ref, up_ref, us_ref, dp_ref, ds_ref,
                out_ref, wg_ref, wu_ref, wd_ref):
    e = pl.program_id(0)

    def unpack(packed_u32, scale_rep_ref, w_scratch, n_words):
        scale_rep = scale_rep_ref.astype(jnp.float32)
        for k in range(8):
            nib = jax.lax.shift_right_logical(
                packed_u32, jnp.uint32(4 * k)).astype(jnp.int32) & 15
            val = _decode_nibbles(nib) * scale_rep
            w_scratch[:, k * n_words:(k + 1) * n_words] = val.astype(jnp.bfloat16)

    unpack(gp_ref[0], gs_ref[0], wg_ref, HID_W)
    unpack(up_ref[0], us_ref[0], wu_ref, HID_W)
    unpack(dp_ref[0], ds_ref[0], wd_ref, INT_W)

    w_e = jnp.sum(jnp.where(tki_ref[...] == e, tkw_ref[...], 0.0),
                  axis=1, keepdims=True)

    x = x_ref[...].astype(jnp.bfloat16)
    dn = (((1,), (1,)), ((), ()))
    g = jax.lax.dot_general(x, wg_ref[...], dn,
                            preferred_element_type=jnp.float32)
    u = jax.lax.dot_general(x, wu_ref[...], dn,
                            preferred_element_type=jnp.float32)
    h = (g * jax.nn.sigmoid(g) * u).astype(jnp.bfloat16)
    d = jax.lax.dot_general(h, wd_ref[...], dn,
                            preferred_element_type=jnp.float32)
    contrib = w_e * d

    @pl.when(e == 0)
    def _():
        out_ref[...] = contrib

    @pl.when(e > 0)
    def _():
        out_ref[...] += contrib


def _perm_inter(a):
    E = a.shape[0]
    return a.reshape(E, INT_W, 8, *a.shape[2:]).swapaxes(1, 2).reshape(
        E, INTER, *a.shape[2:])


_PERM_ROWS = NUM_EXPERTS * INTER
_SC_CHUNK = 32


def _sc_permute_rows(table, idx):
    per_w = _PERM_ROWS // 32

    mesh = plsc.VectorSubcoreMesh(core_axis_name="c", subcore_axis_name="s")

    @functools.partial(
        pl.kernel, mesh=mesh,
        out_type=jax.ShapeDtypeStruct((_PERM_ROWS, HID_W), jnp.int32),
        scratch_types=[
            pltpu.VMEM((_SC_CHUNK,), jnp.int32),
            pltpu.VMEM((_SC_CHUNK, HID_W), jnp.int32),
            pltpu.SemaphoreType.DMA,
        ],
    )
    def gather_kernel(table_hbm, idx_hbm, out_hbm, idx_v, rows_v, sem):
        wid = lax.axis_index("s") * 2 + lax.axis_index("c")
        base = wid * per_w
        for c in range(per_w // _SC_CHUNK):
            off = base + c * _SC_CHUNK
            pltpu.sync_copy(idx_hbm.at[pl.ds(off, _SC_CHUNK)], idx_v)
            pltpu.async_copy(table_hbm.at[idx_v], rows_v, sem).wait()
            pltpu.sync_copy(rows_v, out_hbm.at[pl.ds(off, _SC_CHUNK)])

    return gather_kernel(table, idx)


def _perm_inter_sc(a_packed):
    i32 = jnp.int32
    j = jnp.arange(_PERM_ROWS, dtype=i32)
    e, r = j // INTER, j % INTER
    idx = e * INTER + 8 * (r % INT_W) + r // INT_W
    flat = lax.bitcast_convert_type(a_packed, i32).reshape(_PERM_ROWS, HID_W)
    out = _sc_permute_rows(flat, idx)
    return lax.bitcast_convert_type(out, jnp.uint32).reshape(
        NUM_EXPERTS, INTER, HID_W)


@jax.jit
def kernel(hidden_states, top_k_index, top_k_weights,
           gate_proj_packed, gate_proj_scales,
           up_proj_packed, up_proj_scales,
           down_proj_packed, down_proj_scales):
    xr = hidden_states.reshape(TOKENS, HID_W, 8).swapaxes(1, 2).reshape(
        TOKENS, HIDDEN)

    gp = _perm_inter_sc(gate_proj_packed)
    up = _perm_inter_sc(up_proj_packed)
    gs = jnp.repeat(_perm_inter(gate_proj_scales.transpose(0, 2, 1)), 16,
                    axis=2).astype(jnp.bfloat16)
    us = jnp.repeat(_perm_inter(up_proj_scales.transpose(0, 2, 1)), 16,
                    axis=2).astype(jnp.bfloat16)
    ds = jnp.repeat(down_proj_scales.transpose(0, 2, 1), 16,
                    axis=2).astype(jnp.bfloat16)

    grid = (NUM_EXPERTS,)
    expert_block = lambda s: pl.BlockSpec((1,) + s, lambda e: (e, 0, 0))
    full = lambda s: pl.BlockSpec(s, lambda e: (0, 0))

    out = pl.pallas_call(
        _moe_kernel,
        grid=grid,
        in_specs=[
            full((TOKENS, TOPK)),
            full((TOKENS, TOPK)),
            full((TOKENS, HIDDEN)),
            expert_block((INTER, HID_W)),
            expert_block((INTER, HID_W)),
            expert_block((INTER, HID_W)),
            expert_block((INTER, HID_W)),
            expert_block((HIDDEN, INT_W)),
            expert_block((HIDDEN, INT_W)),
        ],
        out_specs=full((TOKENS, HIDDEN)),
        out_shape=jax.ShapeDtypeStruct((TOKENS, HIDDEN), jnp.float32),
        scratch_shapes=[
            pltpu.VMEM((INTER, HIDDEN), jnp.bfloat16),
            pltpu.VMEM((INTER, HIDDEN), jnp.bfloat16),
            pltpu.VMEM((HIDDEN, INTER), jnp.bfloat16),
        ],
    )(top_k_index, top_k_weights, xr, gp, gs, up, us, down_proj_packed, ds)
    return out


# speedup vs baseline: 1.6819x; 1.0228x over previous
import functools

import jax
import jax.numpy as jnp
from jax import lax
from jax.experimental import pallas as pl
from jax.experimental.pallas import tpu as pltpu
from jax.experimental.pallas import tpu_sc as plsc

NUM_EXPERTS = 8
HIDDEN = 1024
INTER = 1408
GROUP = 128
TOKENS = 2048
TOPK = 2

HID_W = HIDDEN // 8
INT_W = INTER // 8


def _decode_nibbles(nib):
    m = nib & 7
    e = m >> 1
    f = (m & 1).astype(jnp.float32)
    pow2 = (jnp.int32(1) << e).astype(jnp.float32) * 0.5
    mag = jnp.where(e == 0, 0.5 * f, pow2 * (1.0 + 0.5 * f))
    sign = 1.0 - 2.0 * (nib >> 3).astype(jnp.float32)
    return sign * mag


def _moe_kernel(tki_ref, tkw_ref, x_ref,
                gp_ref, gs_ref, up_ref, us_ref, dp_ref, ds_ref,
                out_ref, wg_ref, wu_ref, wd_ref):
    e = pl.program_id(0)

    def unpack(packed_u32, scale_rep_ref, w_scratch, n_words):
        scale_rep = scale_rep_ref.astype(jnp.float32)
        for k in range(8):
            nib = jax.lax.shift_right_logical(
                packed_u32, jnp.uint32(4 * k)).astype(jnp.int32) & 15
            val = _decode_nibbles(nib) * scale_rep
            w_scratch[:, k * n_words:(k + 1) * n_words] = val.astype(jnp.bfloat16)

    unpack(gp_ref[0], gs_ref[0], wg_ref, HID_W)
    unpack(up_ref[0], us_ref[0], wu_ref, HID_W)
    unpack(dp_ref[0], ds_ref[0], wd_ref, INT_W)

    w_e = jnp.sum(jnp.where(tki_ref[...] == e, tkw_ref[...], 0.0),
                  axis=1, keepdims=True)

    x = x_ref[...].astype(jnp.bfloat16)
    dn = (((1,), (1,)), ((), ()))
    g = jax.lax.dot_general(x, wg_ref[...], dn,
                            preferred_element_type=jnp.float32)
    u = jax.lax.dot_general(x, wu_ref[...], dn,
                            preferred_element_type=jnp.float32)
    h = (g * jax.nn.sigmoid(g) * u).astype(jnp.bfloat16)
    d = jax.lax.dot_general(h, wd_ref[...], dn,
                            preferred_element_type=jnp.float32)
    contrib = w_e * d

    @pl.when(e == 0)
    def _():
        out_ref[...] = contrib

    @pl.when(e > 0)
    def _():
        out_ref[...] += contrib


def _perm_inter(a):
    E = a.shape[0]
    return a.reshape(E, INT_W, 8, *a.shape[2:]).swapaxes(1, 2).reshape(
        E, INTER, *a.shape[2:])


_PERM_ROWS = NUM_EXPERTS * INTER
_SC_CHUNK = 32


def _sc_permute_rows(table, idx):
    per_w = _PERM_ROWS // 32

    mesh = plsc.VectorSubcoreMesh(core_axis_name="c", subcore_axis_name="s")

    @functools.partial(
        pl.kernel, mesh=mesh,
        out_type=jax.ShapeDtypeStruct((_PERM_ROWS, HID_W), jnp.int32),
        scratch_types=[
            pltpu.VMEM((per_w,), jnp.int32),
            pltpu.VMEM((per_w, HID_W), jnp.int32),
            pltpu.SemaphoreType.DMA,
        ],
    )
    def gather_kernel(table_hbm, idx_hbm, out_hbm, idx_v, rows_v, sem):
        wid = lax.axis_index("s") * 2 + lax.axis_index("c")
        base = wid * per_w
        pltpu.sync_copy(idx_hbm.at[pl.ds(base, per_w)], idx_v)
        pltpu.async_copy(table_hbm.at[idx_v], rows_v, sem).wait()
        pltpu.sync_copy(rows_v, out_hbm.at[pl.ds(base, per_w)])

    return gather_kernel(table, idx)


def _perm_inter_sc(a_packed):
    i32 = jnp.int32
    j = jnp.arange(_PERM_ROWS, dtype=i32)
    e, r = j // INTER, j % INTER
    idx = e * INTER + 8 * (r % INT_W) + r // INT_W
    flat = lax.bitcast_convert_type(a_packed, i32).reshape(_PERM_ROWS, HID_W)
    out = _sc_permute_rows(flat, idx)
    return lax.bitcast_convert_type(out, jnp.uint32).reshape(
        NUM_EXPERTS, INTER, HID_W)


@jax.jit
def kernel(hidden_states, top_k_index, top_k_weights,
           gate_proj_packed, gate_proj_scales,
           up_proj_packed, up_proj_scales,
           down_proj_packed, down_proj_scales):
    xr = hidden_states.reshape(TOKENS, HID_W, 8).swapaxes(1, 2).reshape(
        TOKENS, HIDDEN)

    gp = _perm_inter_sc(gate_proj_packed)
    up = _perm_inter_sc(up_proj_packed)
    gs = jnp.repeat(_perm_inter(gate_proj_scales.transpose(0, 2, 1)), 16,
                    axis=2).astype(jnp.bfloat16)
    us = jnp.repeat(_perm_inter(up_proj_scales.transpose(0, 2, 1)), 16,
                    axis=2).astype(jnp.bfloat16)
    ds = jnp.repeat(down_proj_scales.transpose(0, 2, 1), 16,
                    axis=2).astype(jnp.bfloat16)

    grid = (NUM_EXPERTS,)
    expert_block = lambda s: pl.BlockSpec((1,) + s, lambda e: (e, 0, 0))
    full = lambda s: pl.BlockSpec(s, lambda e: (0, 0))

    out = pl.pallas_call(
        _moe_kernel,
        grid=grid,
        in_specs=[
            full((TOKENS, TOPK)),
            full((TOKENS, TOPK)),
            full((TOKENS, HIDDEN)),
            expert_block((INTER, HID_W)),
            expert_block((INTER, HID_W)),
            expert_block((INTER, HID_W)),
            expert_block((INTER, HID_W)),
            expert_block((HIDDEN, INT_W)),
            expert_block((HIDDEN, INT_W)),
        ],
        out_specs=full((TOKENS, HIDDEN)),
        out_shape=jax.ShapeDtypeStruct((TOKENS, HIDDEN), jnp.float32),
        scratch_shapes=[
            pltpu.VMEM((INTER, HIDDEN), jnp.bfloat16),
            pltpu.VMEM((INTER, HIDDEN), jnp.bfloat16),
            pltpu.VMEM((HIDDEN, INTER), jnp.bfloat16),
        ],
    )(top_k_index, top_k_weights, xr, gp, gs, up, us, down_proj_packed, ds)
    return out


# speedup vs baseline: 1.7676x; 1.0510x over previous
import functools

import jax
import jax.numpy as jnp
from jax.experimental import pallas as pl
from jax.experimental.pallas import tpu as pltpu

NUM_EXPERTS = 8
HIDDEN = 1024
INTER = 1408
GROUP = 128
TOKENS = 2048
TOPK = 2

HID_W = HIDDEN // 8
INT_W = INTER // 8


def _decode_nibbles(nib):
    m = nib & 7
    e = m >> 1
    f = (m & 1).astype(jnp.float32)
    pow2 = (jnp.int32(1) << e).astype(jnp.float32) * 0.5
    mag = jnp.where(e == 0, 0.5 * f, pow2 * (1.0 + 0.5 * f))
    sign = 1.0 - 2.0 * (nib >> 3).astype(jnp.float32)
    return sign * mag


def _moe_kernel(tki_ref, tkw_ref, x_ref,
                gp_ref, gs_ref, up_ref, us_ref, dp_ref, ds_ref,
                out_ref, wg_ref, wu_ref, wd_ref):
    e = pl.program_id(0)

    def unpack(packed_u32, scale_rep_ref, w_scratch, n_words):
        scale_rep = scale_rep_ref.astype(jnp.float32)
        for k in range(8):
            nib = jax.lax.shift_right_logical(
                packed_u32, jnp.uint32(4 * k)).astype(jnp.int32) & 15
            val = _decode_nibbles(nib) * scale_rep
            w_scratch[:, k * n_words:(k + 1) * n_words] = val.astype(jnp.bfloat16)

    unpack(gp_ref[0], gs_ref[0], wg_ref, HID_W)
    unpack(up_ref[0], us_ref[0], wu_ref, HID_W)
    unpack(dp_ref[0], ds_ref[0], wd_ref, INT_W)

    w_e = jnp.sum(jnp.where(tki_ref[...] == e, tkw_ref[...], 0.0),
                  axis=1, keepdims=True)

    x = x_ref[...].astype(jnp.bfloat16)
    dn = (((1,), (1,)), ((), ()))
    g = jax.lax.dot_general(x, wg_ref[...], dn,
                            preferred_element_type=jnp.float32)
    u = jax.lax.dot_general(x, wu_ref[...], dn,
                            preferred_element_type=jnp.float32)
    h = (g * jax.nn.sigmoid(g) * u).astype(jnp.bfloat16)
    d = jax.lax.dot_general(h, wd_ref[...], dn,
                            preferred_element_type=jnp.float32)
    contrib = w_e * d

    @pl.when(e == 0)
    def _():
        out_ref[...] = contrib

    @pl.when(e > 0)
    def _():
        out_ref[...] += contrib


def _perm_inter(a):
    E = a.shape[0]
    return a.reshape(E, INT_W, 8, *a.shape[2:]).swapaxes(1, 2).reshape(
        E, INTER, *a.shape[2:])


@jax.jit
def kernel(hidden_states, top_k_index, top_k_weights,
           gate_proj_packed, gate_proj_scales,
           up_proj_packed, up_proj_scales,
           down_proj_packed, down_proj_scales):
    xr = hidden_states.reshape(TOKENS, HID_W, 8).swapaxes(1, 2).reshape(
        TOKENS, HIDDEN)

    gp = _perm_inter(gate_proj_packed)
    up = _perm_inter(up_proj_packed)
    gs = jnp.repeat(_perm_inter(gate_proj_scales.transpose(0, 2, 1)), 16,
                    axis=2).astype(jnp.bfloat16)
    us = jnp.repeat(_perm_inter(up_proj_scales.transpose(0, 2, 1)), 16,
                    axis=2).astype(jnp.bfloat16)
    ds = jnp.repeat(down_proj_scales.transpose(0, 2, 1), 16,
                    axis=2).astype(jnp.bfloat16)

    grid = (NUM_EXPERTS,)
    expert_block = lambda s: pl.BlockSpec((1,) + s, lambda e: (e, 0, 0))
    full = lambda s: pl.BlockSpec(s, lambda e: (0, 0))

    out = pl.pallas_call(
        _moe_kernel,
        grid=grid,
        in_specs=[
            full((TOKENS, TOPK)),
            full((TOKENS, TOPK)),
            full((TOKENS, HIDDEN)),
            expert_block((INTER, HID_W)),
            expert_block((INTER, HID_W)),
            expert_block((INTER, HID_W)),
            expert_block((INTER, HID_W)),
            expert_block((HIDDEN, INT_W)),
            expert_block((HIDDEN, INT_W)),
        ],
        out_specs=full((TOKENS, HIDDEN)),
        out_shape=jax.ShapeDtypeStruct((TOKENS, HIDDEN), jnp.float32),
        scratch_shapes=[
            pltpu.VMEM((INTER, HIDDEN), jnp.bfloat16),
            pltpu.VMEM((INTER, HIDDEN), jnp.bfloat16),
            pltpu.VMEM((HIDDEN, INTER), jnp.bfloat16),
        ],
    )(top_k_index, top_k_weights, xr, gp, gs, up, us, down_proj_packed, ds)
    return out
